# baseline (speedup 1.0000x reference)
import functools

import jax
import jax.numpy as jnp
from jax import lax
from jax.experimental import pallas as pl
from jax.experimental.pallas import tpu as pltpu
from jax.experimental.pallas import tpu_sc as plsc

_E = 320000
_N = 10000
_D = 128
_NC = 2
_NS = 16
_NW = _NC * _NS
_EPW = _E // _NW
_C = 128
_EPAD = 10112
_NCH = _EPAD // _C
_FB = _D // 16


_PP = 129


def _body(edge_hbm, pos_hbm, sig_hbm, out_hbm,
          idx_a, idx_b, rows_a, rows_b, acc, psum, sig_v, pos_sh,
          sem_a0, sem_b0, sem_a1, sem_b1, sem_a2, sem_b2):
    sid = lax.axis_index("s")
    wid = sid * _NC + lax.axis_index("c")
    seg = _N // _NS
    pltpu.sync_copy(pos_hbm.at[pl.ds(sid * seg, seg)],
                    pos_sh.at[pl.ds(sid * seg, seg)])
    pltpu.sync_copy(edge_hbm.at[0, wid], idx_a)
    pltpu.sync_copy(edge_hbm.at[1, wid], idx_b)
    pltpu.sync_copy(sig_hbm, sig_v)
    plsc.subcore_barrier()

    sems = ((sem_a0, sem_b0), (sem_a1, sem_b1), (sem_a2, sem_b2))

    def start(chunk, slot):
        sa, sb = sems[slot]
        pltpu.make_async_copy(
            pos_sh.at[idx_a.at[chunk]], rows_a.at[slot], sa).start()
        pltpu.make_async_copy(
            pos_sh.at[idx_b.at[chunk]], rows_b.at[slot], sb).start()

    def wait(chunk, slot):
        sa, sb = sems[slot]
        pltpu.make_async_copy(
            pos_sh.at[idx_a.at[chunk]], rows_a.at[slot], sa).wait()
        pltpu.make_async_copy(
            pos_sh.at[idx_b.at[chunk]], rows_b.at[slot], sb).wait()

    neg_sig = sig_v[...]
    colv = lax.iota(jnp.int32, 16) * _PP

    def compute(chunk, slot):
        ra = rows_a.at[slot]
        rb = rows_b.at[slot]

        @plsc.parallel_loop(0, _C, unroll=8)
        def edge_body(e):
            sq = []
            for f in range(_D // 32):
                a = plsc.bitcast(ra[e, pl.ds(16 * f, 16)], jnp.bfloat16)
                b = plsc.bitcast(rb[e, pl.ds(16 * f, 16)], jnp.bfloat16)
                d = a - b
                sq.append(d * d)
            s0 = (sq[0] + sq[1]) + (sq[2] + sq[3])
            s0l, s0u = plsc.unpack(s0, format=plsc.PackFormat.INTERLEAVED)
            plsc.store_scatter(psum, [colv + e], s0l + s0u)

        @plsc.parallel_loop(0, _C // 16, unroll=4)
        def red_body(g):
            t = [psum[pl.ds(k * _PP + 16 * g, 16)] for k in range(16)]
            while len(t) > 1:
                t = [t[2 * i] + t[2 * i + 1] for i in range(len(t) // 2)]
            acc[pl.ds(chunk * _C + 16 * g, 16)] = t[0]

    for p in range(3):
        start(p, p)

    def loop_body(j, carry):
        i = 3 * j
        for p in range(3):
            wait(i + p, p)
            compute(i + p, p)

            @pl.when(i + p + 3 < _NCH)
            def _():
                start(i + p + 3, p)

        return carry

    lax.fori_loop(0, _NCH // 3, loop_body, 0)
    for p in range(_NCH % 3):
        wait((_NCH // 3) * 3 + p, p)
        compute((_NCH // 3) * 3 + p, p)

    @plsc.parallel_loop(0, _EPAD // 16, unroll=4)
    def exp_pass(m):
        v = acc[pl.ds(16 * m, 16)]
        acc[pl.ds(16 * m, 16)] = jnp.exp(neg_sig * v)

    pltpu.sync_copy(acc.at[pl.ds(0, _EPW)], out_hbm.at[wid])


@functools.partial(jax.jit, static_argnames=())
def kernel(edge_idxs, pos, inv_sigma):
    edge_r = jnp.pad(
        edge_idxs.reshape(2, _NW, _EPW),
        ((0, 0), (0, 0), (0, _EPAD - _EPW))).reshape(2, _NW, _NCH, _C)
    pos = lax.bitcast_convert_type(
        pos.astype(jnp.bfloat16).reshape(_N, _D // 2, 2), jnp.int32)
    sig = jnp.broadcast_to(-inv_sigma.astype(jnp.float32), (16,))
    mesh = plsc.VectorSubcoreMesh(
        core_axis_name="c", subcore_axis_name="s",
        num_cores=_NC, num_subcores=_NS)
    out = pl.kernel(
        _body,
        out_type=jax.ShapeDtypeStruct((_NW, _EPW), jnp.float32),
        mesh=mesh,
        compiler_params=pltpu.CompilerParams(
            needs_layout_passes=False, use_tc_tiling_on_sc=False),
        scratch_types=[
            pltpu.VMEM((_NCH, _C), jnp.int32),
            pltpu.VMEM((_NCH, _C), jnp.int32),
            pltpu.VMEM((3, _C, _D // 2), jnp.int32),
            pltpu.VMEM((3, _C, _D // 2), jnp.int32),
            pltpu.VMEM((_EPAD,), jnp.float32),
            pltpu.VMEM((16 * _PP,), jnp.float32),
            pltpu.VMEM((16,), jnp.float32),
            pltpu.VMEM_SHARED((_N, _D // 2), jnp.int32),
            pltpu.SemaphoreType.DMA,
            pltpu.SemaphoreType.DMA,
            pltpu.SemaphoreType.DMA,
            pltpu.SemaphoreType.DMA,
            pltpu.SemaphoreType.DMA,
            pltpu.SemaphoreType.DMA,
        ],
    )(edge_r, pos, sig)
    return out.reshape(_E)


# speedup vs baseline: 1.0581x; 1.0581x over previous
import functools

import jax
import jax.numpy as jnp
from jax import lax
from jax.experimental import pallas as pl
from jax.experimental.pallas import tpu as pltpu
from jax.experimental.pallas import tpu_sc as plsc

_E = 320000
_N = 10000
_D = 128
_NC = 2
_NS = 16
_NW = _NC * _NS
_EPW = _E // _NW
_C = 80
_NCH = _EPW // _C
_FB = _D // 16


_PP = 81


def _body(edge_hbm, pos_hbm, sig_hbm, out_hbm,
          idx_a, idx_b, rows_a, rows_b, acc, psum, sig_v, pos_sh,
          sem_a0, sem_b0, sem_a1, sem_b1, sem_a2, sem_b2, sem_a3, sem_b3):
    sid = lax.axis_index("s")
    wid = sid * _NC + lax.axis_index("c")
    seg = _N // _NS
    cp_t = pltpu.make_async_copy(pos_hbm.at[pl.ds(sid * seg, seg)],
                                 pos_sh.at[pl.ds(sid * seg, seg)], sem_a0)
    cp_t.start()
    cp_ia = pltpu.make_async_copy(edge_hbm.at[0, wid], idx_a, sem_b0)
    cp_ia.start()
    cp_ib = pltpu.make_async_copy(edge_hbm.at[1, wid], idx_b, sem_a1)
    cp_ib.start()
    pltpu.sync_copy(sig_hbm, sig_v)
    cp_t.wait()
    cp_ia.wait()
    cp_ib.wait()
    plsc.subcore_barrier()

    sems = ((sem_a0, sem_b0), (sem_a1, sem_b1),
            (sem_a2, sem_b2), (sem_a3, sem_b3))

    def start(chunk, slot):
        sa, sb = sems[slot]
        pltpu.make_async_copy(
            pos_sh.at[idx_a.at[chunk]], rows_a.at[slot], sa).start()
        pltpu.make_async_copy(
            pos_sh.at[idx_b.at[chunk]], rows_b.at[slot], sb).start()

    def wait(chunk, slot):
        sa, sb = sems[slot]
        pltpu.make_async_copy(
            pos_sh.at[idx_a.at[chunk]], rows_a.at[slot], sa).wait()
        pltpu.make_async_copy(
            pos_sh.at[idx_b.at[chunk]], rows_b.at[slot], sb).wait()

    neg_sig = sig_v[...]
    colv = lax.iota(jnp.int32, 16) * _PP

    def compute(chunk, slot):
        ra = rows_a.at[slot]
        rb = rows_b.at[slot]

        @plsc.parallel_loop(0, _C, unroll=8)
        def edge_body(e):
            sq = []
            for f in range(_D // 32):
                a = plsc.bitcast(ra[e, pl.ds(16 * f, 16)], jnp.bfloat16)
                b = plsc.bitcast(rb[e, pl.ds(16 * f, 16)], jnp.bfloat16)
                d = a - b
                sq.append(d * d)
            s0 = (sq[0] + sq[1]) + (sq[2] + sq[3])
            s0l, s0u = plsc.unpack(s0, format=plsc.PackFormat.INTERLEAVED)
            plsc.store_scatter(psum, [colv + e], s0l + s0u)

        @plsc.parallel_loop(0, _C // 16, unroll=5)
        def red_body(g):
            t = [psum[pl.ds(k * _PP + 16 * g, 16)] for k in range(16)]
            while len(t) > 1:
                t = [t[2 * i] + t[2 * i + 1] for i in range(len(t) // 2)]
            acc[pl.ds(chunk * _C + 16 * g, 16)] = t[0]

    for p in range(4):
        start(p, p)

    def loop_body(j, carry):
        i = 4 * j
        for p in range(4):
            wait(i + p, p)
            compute(i + p, p)

            @pl.when(i + p + 4 < _NCH)
            def _():
                start(i + p + 4, p)

        return carry

    lax.fori_loop(0, (_NCH - 1) // 4, loop_body, 0)
    wait(_NCH - 1, 0)
    compute(_NCH - 1, 0)

    @plsc.parallel_loop(0, _EPW // 16, unroll=4)
    def exp_pass(m):
        v = acc[pl.ds(16 * m, 16)]
        acc[pl.ds(16 * m, 16)] = jnp.exp(neg_sig * v)

    pltpu.sync_copy(acc, out_hbm.at[wid])


@functools.partial(jax.jit, static_argnames=())
def kernel(edge_idxs, pos, inv_sigma):
    edge_r = edge_idxs.reshape(2, _NW, _NCH, _C)
    pos = lax.bitcast_convert_type(
        pos.astype(jnp.bfloat16).reshape(_N, _D // 2, 2), jnp.int32)
    sig = jnp.broadcast_to(-inv_sigma.astype(jnp.float32), (16,))
    mesh = plsc.VectorSubcoreMesh(
        core_axis_name="c", subcore_axis_name="s",
        num_cores=_NC, num_subcores=_NS)
    out = pl.kernel(
        _body,
        out_type=jax.ShapeDtypeStruct((_NW, _EPW), jnp.float32),
        mesh=mesh,
        compiler_params=pltpu.CompilerParams(
            needs_layout_passes=False, use_tc_tiling_on_sc=False),
        scratch_types=[
            pltpu.VMEM((_NCH, _C), jnp.int32),
            pltpu.VMEM((_NCH, _C), jnp.int32),
            pltpu.VMEM((4, _C, _D // 2), jnp.int32),
            pltpu.VMEM((4, _C, _D // 2), jnp.int32),
            pltpu.VMEM((_EPW,), jnp.float32),
            pltpu.VMEM((16 * _PP,), jnp.float32),
            pltpu.VMEM((16,), jnp.float32),
            pltpu.VMEM_SHARED((_N, _D // 2), jnp.int32),
            pltpu.SemaphoreType.DMA,
            pltpu.SemaphoreType.DMA,
            pltpu.SemaphoreType.DMA,
            pltpu.SemaphoreType.DMA,
            pltpu.SemaphoreType.DMA,
            pltpu.SemaphoreType.DMA,
            pltpu.SemaphoreType.DMA,
            pltpu.SemaphoreType.DMA,
        ],
    )(edge_r, pos, sig)
    return out.reshape(_E)
